# baseline (speedup 1.0000x reference)
import functools

import jax
import jax.numpy as jnp
from jax import lax
from jax.experimental import pallas as pl
from jax.experimental.pallas import tpu as pltpu
from jax.experimental.pallas import tpu_sc as plsc

_VOCAB = 1000000
_EMBED_DIM = 32
_BATCH = 16384

_info = plsc.get_sparse_core_info()
_NC, _NS = _info.num_cores, _info.num_subcores
_NW = _NC * _NS
_B_PER_W = _BATCH // _NW

_mesh = plsc.VectorSubcoreMesh(core_axis_name="c", subcore_axis_name="s")


@functools.partial(
    pl.kernel,
    mesh=_mesh,
    out_type=jax.ShapeDtypeStruct((_BATCH, _EMBED_DIM), jnp.float32),
    scratch_types=[
        pltpu.VMEM((_B_PER_W,), jnp.int32),
        pltpu.VMEM((_B_PER_W, _EMBED_DIM), jnp.float32),
        pltpu.SemaphoreType.DMA,
    ],
    compiler_params=pltpu.CompilerParams(use_tc_tiling_on_sc=False),
)
def _gather_kernel(idx_hbm, tab_hbm, out_hbm, idx_v, rows_v, sem):
    wid = lax.axis_index("s") * _NC + lax.axis_index("c")
    base = wid * _B_PER_W
    pltpu.sync_copy(idx_hbm.at[pl.ds(base, _B_PER_W)], idx_v)
    pltpu.async_copy(tab_hbm.at[idx_v], rows_v, sem).wait()
    pltpu.sync_copy(rows_v, out_hbm.at[pl.ds(base, _B_PER_W)])


def kernel(node_feats, table):
    return _gather_kernel(node_feats.astype(jnp.int32), table)


# speedup vs baseline: 3.8004x; 3.8004x over previous
import functools

import jax
import jax.numpy as jnp
from jax import lax
from jax.experimental import pallas as pl
from jax.experimental.pallas import tpu as pltpu
from jax.experimental.pallas import tpu_sc as plsc

_VOCAB = 1000000
_EMBED_DIM = 32
_BATCH = 16384

_info = plsc.get_sparse_core_info()
_NC, _NS = _info.num_cores, _info.num_subcores
_NW = _NC * _NS
_B_PER_W = _BATCH // _NW
_LANES = _info.num_lanes
_GRP = 8

_mesh = plsc.VectorSubcoreMesh(core_axis_name="c", subcore_axis_name="s")


@functools.partial(
    pl.kernel,
    mesh=_mesh,
    out_type=jax.ShapeDtypeStruct((_EMBED_DIM, _BATCH), jnp.float32),
    scratch_types=[
        pltpu.VMEM((_B_PER_W,), jnp.int32),
        pltpu.VMEM((2, _GRP, _EMBED_DIM, 128), jnp.float32),
        pltpu.VMEM((_EMBED_DIM, _B_PER_W), jnp.float32),
        pltpu.SemaphoreType.DMA,
        pltpu.SemaphoreType.DMA,
    ],
    compiler_params=pltpu.CompilerParams(
        use_tc_tiling_on_sc=True, needs_layout_passes=False
    ),
)
def _gather_kernel(idx_hbm, tab_hbm, out_hbm, idx_v, ring, vals, sem0, sem1):
    wid = lax.axis_index("s") * _NC + lax.axis_index("c")
    base = wid * _B_PER_W
    pltpu.sync_copy(idx_hbm.at[pl.ds(base, _B_PER_W)], idx_v)

    d_lo = lax.iota(jnp.int32, _LANES)
    d_hi = d_lo + _LANES
    sems = [sem0, sem1]

    def fire(p, col, off):
        for j in range(_GRP):
            pltpu.async_copy(
                tab_hbm.at[:, pl.ds(col[off + j] * 128, 128)],
                ring.at[p, j],
                sems[p],
            )

    def extract(p, lane_vec, off, i0):
        for j in range(_GRP):
            pltpu.make_async_copy(
                tab_hbm.at[:, pl.ds(0, 128)], ring.at[p, j], sems[p]
            ).wait()
        for j in range(_GRP):
            lv = jnp.full((_LANES,), lane_vec[off + j], jnp.int32)
            buf = ring.at[p, j]
            v_lo = plsc.load_gather(buf, [d_lo, lv])
            v_hi = plsc.load_gather(buf, [d_hi, lv])
            i_vec = jnp.full((_LANES,), i0 + j, jnp.int32)
            plsc.store_scatter(vals, [d_lo, i_vec], v_lo)
            plsc.store_scatter(vals, [d_hi, i_vec], v_hi)

    def body(k, lane_prev):
        vec = idx_v[pl.ds(k * _LANES, _LANES)]
        col = lax.shift_right_logical(vec, 7)
        lane = lax.bitwise_and(vec, 127)
        fire(0, col, 0)

        @pl.when(k > 0)
        def _():
            extract(1, lane_prev, _GRP, (2 * k - 1) * _GRP)

        fire(1, col, _GRP)
        extract(0, lane, 0, 2 * k * _GRP)
        return lane

    lane_last = lax.fori_loop(0, _B_PER_W // _LANES, body, d_lo)
    extract(1, lane_last, _GRP, _B_PER_W - _GRP)

    pltpu.sync_copy(vals, out_hbm.at[:, pl.ds(base, _B_PER_W)])


def kernel(node_feats, table):
    out_t = _gather_kernel(node_feats.astype(jnp.int32), jnp.transpose(table))
    return jnp.transpose(out_t)
